# baseline (device time: 13563 ns/iter reference)
import jax
import jax.numpy as jnp
from jax import lax
from jax.experimental import pallas as pl
from jax.experimental.pallas import tpu as pltpu

N_CHUNKS = 8


def kernel(x):
    _, m, n = x.shape
    half = n // 2
    rows = m // N_CHUNKS

    def body(
        x_ref,
        out_ref,
        send_q,
        recv_q,
        scale_send,
        scale_recv,
        send_sems,
        recv_sems,
        scale_sems,
    ):
        my_x = lax.axis_index("x")
        my_y = lax.axis_index("y")
        my_z = lax.axis_index("z")
        partner = (my_x, my_y, 1 - my_z)

        barrier = pltpu.get_barrier_semaphore()
        pl.semaphore_signal(
            barrier, inc=1, device_id=partner,
            device_id_type=pl.DeviceIdType.MESH,
        )

        def go(send_lo, keep_lo):
            s = jnp.max(jnp.abs(x_ref[0, :, send_lo:send_lo + half]))
            s = jnp.maximum(s, 1e-30)
            scale_send[:, :] = jnp.full((8, 128), s / 127.0, jnp.float32)
            inv = 127.0 / s
            send_q[0:rows, :] = jnp.round(
                x_ref[0, 0:rows, send_lo:send_lo + half] * inv
            ).astype(jnp.int8)

            pl.semaphore_wait(barrier, 1)
            scale_rdma = pltpu.make_async_remote_copy(
                src_ref=scale_send,
                dst_ref=scale_recv,
                send_sem=scale_sems.at[0],
                recv_sem=scale_sems.at[1],
                device_id=partner,
                device_id_type=pl.DeviceIdType.MESH,
            )
            scale_rdma.start()

            rdmas = []
            for k in range(N_CHUNKS):
                r0, r1 = k * rows, (k + 1) * rows
                if k > 0:
                    send_q[r0:r1, :] = jnp.round(
                        x_ref[0, r0:r1, send_lo:send_lo + half] * inv
                    ).astype(jnp.int8)
                rdma = pltpu.make_async_remote_copy(
                    src_ref=send_q.at[r0:r1, :],
                    dst_ref=recv_q.at[r0:r1, :],
                    send_sem=send_sems.at[k],
                    recv_sem=recv_sems.at[k],
                    device_id=partner,
                    device_id_type=pl.DeviceIdType.MESH,
                )
                rdma.start()
                rdmas.append(rdma)

            scale_rdma.wait_recv()
            peer_scale = scale_recv[0, 0]
            for k in range(N_CHUNKS):
                r0, r1 = k * rows, (k + 1) * rows
                rdmas[k].wait_recv()
                out_ref[r0:r1, :] = (
                    x_ref[0, r0:r1, keep_lo:keep_lo + half]
                    + recv_q[r0:r1, :].astype(jnp.float32) * peer_scale
                )
            scale_rdma.wait_send()
            for k in range(N_CHUNKS):
                rdmas[k].wait_send()

        @pl.when(my_z == 0)
        def _():
            go(half, 0)

        @pl.when(my_z == 1)
        def _():
            go(0, half)

    return pl.pallas_call(
        body,
        out_shape=jax.ShapeDtypeStruct((m, half), x.dtype),
        in_specs=[pl.BlockSpec(memory_space=pltpu.VMEM)],
        out_specs=pl.BlockSpec(memory_space=pltpu.VMEM),
        scratch_shapes=[
            pltpu.VMEM((m, half), jnp.int8),
            pltpu.VMEM((m, half), jnp.int8),
            pltpu.VMEM((8, 128), jnp.float32),
            pltpu.VMEM((8, 128), jnp.float32),
            pltpu.SemaphoreType.DMA((N_CHUNKS,)),
            pltpu.SemaphoreType.DMA((N_CHUNKS,)),
            pltpu.SemaphoreType.DMA((2,)),
        ],
        compiler_params=pltpu.CompilerParams(collective_id=0),
    )(x)


# device time: 13535 ns/iter; 1.0021x vs baseline; 1.0021x over previous
import jax
import jax.numpy as jnp
from jax import lax
from jax.experimental import pallas as pl
from jax.experimental.pallas import tpu as pltpu

N_CHUNKS = 8


def kernel(x):
    _, m, n = x.shape
    half = n // 2
    rows = m // N_CHUNKS

    def body(
        x_ref,
        out_ref,
        send_q,
        recv_q,
        scale_send,
        scale_recv,
        send_sems,
        recv_sems,
        scale_sems,
    ):
        my_x = lax.axis_index("x")
        my_y = lax.axis_index("y")
        my_z = lax.axis_index("z")
        partner = (my_x, my_y, 1 - my_z)

        barrier = pltpu.get_barrier_semaphore()
        pl.semaphore_signal(
            barrier, inc=1, device_id=partner,
            device_id_type=pl.DeviceIdType.MESH,
        )
        pl.semaphore_wait(barrier, 1)

        def go(send_lo, keep_lo):
            s = jnp.max(jnp.abs(x_ref[0, :, send_lo:send_lo + half]))
            s = jnp.maximum(s, 1e-30)
            scale_send[:, :] = jnp.full((8, 128), s / 127.0, jnp.float32)
            scale_rdma = pltpu.make_async_remote_copy(
                src_ref=scale_send,
                dst_ref=scale_recv,
                send_sem=scale_sems.at[0],
                recv_sem=scale_sems.at[1],
                device_id=partner,
                device_id_type=pl.DeviceIdType.MESH,
            )
            scale_rdma.start()

            inv = 127.0 / s
            rdmas = []
            for k in range(N_CHUNKS):
                r0, r1 = k * rows, (k + 1) * rows
                send_q[r0:r1, :] = jnp.round(
                    x_ref[0, r0:r1, send_lo:send_lo + half] * inv
                ).astype(jnp.int8)
                rdma = pltpu.make_async_remote_copy(
                    src_ref=send_q.at[r0:r1, :],
                    dst_ref=recv_q.at[r0:r1, :],
                    send_sem=send_sems.at[k],
                    recv_sem=recv_sems.at[k],
                    device_id=partner,
                    device_id_type=pl.DeviceIdType.MESH,
                )
                rdma.start()
                rdmas.append(rdma)

            scale_rdma.wait_recv()
            peer_scale = scale_recv[0, 0]
            for k in range(N_CHUNKS):
                r0, r1 = k * rows, (k + 1) * rows
                rdmas[k].wait_recv()
                out_ref[r0:r1, :] = (
                    x_ref[0, r0:r1, keep_lo:keep_lo + half]
                    + recv_q[r0:r1, :].astype(jnp.float32) * peer_scale
                )
            scale_rdma.wait_send()
            for k in range(N_CHUNKS):
                rdmas[k].wait_send()

        @pl.when(my_z == 0)
        def _():
            go(half, 0)

        @pl.when(my_z == 1)
        def _():
            go(0, half)

    return pl.pallas_call(
        body,
        out_shape=jax.ShapeDtypeStruct((m, half), x.dtype),
        in_specs=[pl.BlockSpec(memory_space=pltpu.VMEM)],
        out_specs=pl.BlockSpec(memory_space=pltpu.VMEM),
        scratch_shapes=[
            pltpu.VMEM((m, half), jnp.int8),
            pltpu.VMEM((m, half), jnp.int8),
            pltpu.VMEM((8, 128), jnp.float32),
            pltpu.VMEM((8, 128), jnp.float32),
            pltpu.SemaphoreType.DMA((N_CHUNKS,)),
            pltpu.SemaphoreType.DMA((N_CHUNKS,)),
            pltpu.SemaphoreType.DMA((2,)),
        ],
        compiler_params=pltpu.CompilerParams(collective_id=0),
    )(x)
